# Initial kernel scaffold; baseline (speedup 1.0000x reference)
import jax
import jax.numpy as jnp
from jax.experimental import pallas as pl


def kernel(inputs):
    raise NotImplementedError("write your pallas kernel here")


# speedup vs baseline: 1.5605x; 1.5605x over previous
import functools

import jax
import jax.numpy as jnp
from jax import lax
from jax.experimental import pallas as pl
from jax.experimental.pallas import tpu as pltpu
from jax.experimental.pallas import tpu_sc as plsc

PATHS = 16
N_TOKENS = 16384
D_MODEL = 2048

NC = 2
NS = 16
NW = NC * NS
B_PER_W = N_TOKENS // NW
CHUNK = 16
NCHUNK = B_PER_W // CHUNK
NBUF = 2


def _dispatch_body(inputs_hbm, perm_hbm, out_hbm,
                   idx_v, rows0, rows1, g0, g1, s0, s1):
  wid = lax.axis_index("s") * NC + lax.axis_index("c")
  row_base = wid * B_PER_W
  chunk_base = wid * NCHUNK

  pltpu.sync_copy(perm_hbm.at[pl.ds(chunk_base, NCHUNK)], idx_v)

  bufs = (rows0, rows1)
  gsems = (g0, g1)
  ssems = (s0, s1)

  def start_gather(k, b):
    pltpu.async_copy(inputs_hbm.at[idx_v.at[k]], bufs[b], gsems[b])

  for b in range(NBUF):
    start_gather(b, b)

  @pl.loop(0, NCHUNK, step=NBUF)
  def _(j):
    for b in range(NBUF):
      k = j + b
      pltpu.make_async_copy(inputs_hbm.at[idx_v.at[k]], bufs[b],
                            gsems[b]).wait()
      store = pltpu.async_copy(
          bufs[b], out_hbm.at[pl.ds(row_base + k * CHUNK, CHUNK)], ssems[b])
      store.wait()

      @pl.when(k + NBUF < NCHUNK)
      def _():
        start_gather(k + NBUF, b)


@jax.jit
def _dispatch(inputs, perm2d):
  mesh = plsc.VectorSubcoreMesh(
      core_axis_name="c", subcore_axis_name="s", num_cores=NC,
      num_subcores=NS)
  f = pl.kernel(
      _dispatch_body,
      out_type=jax.ShapeDtypeStruct((N_TOKENS, D_MODEL), jnp.float32),
      mesh=mesh,
      scratch_types=[
          pltpu.VMEM((NCHUNK, CHUNK), jnp.int32),
          pltpu.VMEM((CHUNK, D_MODEL), jnp.float32),
          pltpu.VMEM((CHUNK, D_MODEL), jnp.float32),
          pltpu.SemaphoreType.DMA,
          pltpu.SemaphoreType.DMA,
          pltpu.SemaphoreType.DMA,
          pltpu.SemaphoreType.DMA,
      ],
  )
  return f(inputs, perm2d)


def kernel(inputs):
  n = inputs.shape[0]
  gate_key = jax.random.fold_in(jax.random.key(42), 0)
  score = jax.random.normal(gate_key, (n, PATHS), dtype=jnp.float32)
  route_idx = jnp.argmax(score, axis=1)
  perm = jnp.argsort(route_idx, stable=True)
  counts = jnp.bincount(route_idx, length=PATHS)
  perm2d = perm.astype(jnp.int32).reshape(N_TOKENS // CHUNK, CHUNK)
  dispatched = _dispatch(inputs, perm2d)
  return dispatched, route_idx, counts
